# Initial kernel scaffold; baseline (speedup 1.0000x reference)
import jax
import jax.numpy as jnp
from jax.experimental import pallas as pl


def kernel(user_ids, item_ids, item_features, user_table, item_table, W1, b1, W2, b2, W3, b3, W4, b4):
    raise NotImplementedError("write your pallas kernel here")


# speedup vs baseline: 1.6261x; 1.6261x over previous
import functools

import jax
import jax.numpy as jnp
from jax import lax
from jax.experimental import pallas as pl
from jax.experimental.pallas import tpu as pltpu
from jax.experimental.pallas import tpu_sc as plsc

B = 16384
ED = 128
NF = 128

NC = 2
NS = 16
NW = NC * NS
CHUNK = 128
N_CHUNK = B // NW // CHUNK
N_IDX_ROWS = B // CHUNK


def _sc_gather_body(uid_hbm, iid_hbm, utab_hbm, itab_hbm,
                    uout_hbm, iout_hbm, idx_v, rows_v, sem):
    wid = lax.axis_index("s") * NC + lax.axis_index("c")
    r0 = wid * N_CHUNK

    pltpu.sync_copy(uid_hbm.at[pl.ds(r0, N_CHUNK)], idx_v)
    cps = [pltpu.async_copy(utab_hbm.at[idx_v.at[j]], rows_v.at[j], sem)
           for j in range(N_CHUNK)]
    for cp in cps:
        cp.wait()
    pltpu.sync_copy(rows_v, uout_hbm.at[pl.ds(r0, N_CHUNK)])

    pltpu.sync_copy(iid_hbm.at[pl.ds(r0, N_CHUNK)], idx_v)
    cps = [pltpu.async_copy(itab_hbm.at[idx_v.at[j]], rows_v.at[j], sem)
           for j in range(N_CHUNK)]
    for cp in cps:
        cp.wait()
    pltpu.sync_copy(rows_v, iout_hbm.at[pl.ds(r0, N_CHUNK)])


def _sc_gather(user_ids2d, item_ids2d, user_table, item_table):
    mesh = plsc.VectorSubcoreMesh(core_axis_name="c", subcore_axis_name="s",
                                  num_cores=NC, num_subcores=NS)
    out_t = jax.ShapeDtypeStruct((N_IDX_ROWS, CHUNK, ED), jnp.float32)
    f = pl.kernel(
        _sc_gather_body,
        out_type=(out_t, out_t),
        mesh=mesh,
        scratch_types=[
            pltpu.VMEM((N_CHUNK, CHUNK), jnp.int32),
            pltpu.VMEM((N_CHUNK, CHUNK, ED), jnp.float32),
            pltpu.SemaphoreType.DMA,
        ],
    )
    return f(user_ids2d, item_ids2d, user_table, item_table)


def _mlp_body(x_ref, u_ref, i_ref, w1_ref, b1_ref, w2_ref, b2_ref,
              w3_ref, b3_ref, w4_ref, b4_ref, o_ref):
    hp = jax.lax.Precision.HIGHEST
    x = x_ref[...]
    h = jnp.maximum(
        jnp.dot(x, w1_ref[...], precision=hp,
                preferred_element_type=jnp.float32) + b1_ref[...], 0.0)
    c = jnp.dot(h, w2_ref[...], precision=hp,
                preferred_element_type=jnp.float32) + b2_ref[...]
    acc = (jnp.dot(u_ref[...], w3_ref[0:ED, :], precision=hp,
                   preferred_element_type=jnp.float32)
           + jnp.dot(i_ref[...], w3_ref[ED:2 * ED, :], precision=hp,
                     preferred_element_type=jnp.float32)
           + jnp.dot(c, w3_ref[2 * ED:3 * ED, :], precision=hp,
                     preferred_element_type=jnp.float32)
           + b3_ref[...])
    p = jnp.maximum(acc, 0.0)
    z = jnp.dot(p, w4_ref[...], precision=hp,
                preferred_element_type=jnp.float32) + b4_ref[...]
    o_ref[...] = jax.nn.sigmoid(z)


def _mlp(x, u, i, W1, b1, W2, b2, W3, b3, W4, b4, bs=2048):
    nblk = B // bs
    row_blk = lambda idx: (idx, 0)
    whole = lambda idx: (0, 0)
    return pl.pallas_call(
        _mlp_body,
        grid=(nblk,),
        in_specs=[
            pl.BlockSpec((bs, NF), row_blk),
            pl.BlockSpec((bs, ED), row_blk),
            pl.BlockSpec((bs, ED), row_blk),
            pl.BlockSpec((NF, ED), whole),
            pl.BlockSpec((1, ED), whole),
            pl.BlockSpec((ED, ED), whole),
            pl.BlockSpec((1, ED), whole),
            pl.BlockSpec((3 * ED, ED), whole),
            pl.BlockSpec((1, ED), whole),
            pl.BlockSpec((ED, 1), whole),
            pl.BlockSpec((1, 1), whole),
        ],
        out_specs=pl.BlockSpec((bs, 1), row_blk),
        out_shape=jax.ShapeDtypeStruct((B, 1), jnp.float32),
    )(x, u, i, W1, b1.reshape(1, ED), W2, b2.reshape(1, ED),
      W3, b3.reshape(1, ED), W4, b4.reshape(1, 1))


def kernel(user_ids, item_ids, item_features, user_table, item_table,
           W1, b1, W2, b2, W3, b3, W4, b4):
    uid2 = user_ids.astype(jnp.int32).reshape(N_IDX_ROWS, CHUNK)
    iid2 = item_ids.astype(jnp.int32).reshape(N_IDX_ROWS, CHUNK)
    u3, i3 = _sc_gather(uid2, iid2, user_table, item_table)
    u = u3.reshape(B, ED)
    i = i3.reshape(B, ED)
    return _mlp(item_features, u, i, W1, b1, W2, b2, W3, b3, W4, b4)


# speedup vs baseline: 2.8423x; 1.7479x over previous
import functools

import jax
import jax.numpy as jnp
from jax import lax
from jax.experimental import pallas as pl
from jax.experimental.pallas import tpu as pltpu
from jax.experimental.pallas import tpu_sc as plsc

B = 16384
ED = 128
NF = 128

NC = 2
NS = 16
NW = NC * NS
CHUNK = 128
N_CHUNK = B // NW // CHUNK
N_IDX_ROWS = B // CHUNK


def _sc_gather_body(uid_hbm, iid_hbm, utab_hbm, itab_hbm,
                    uout_hbm, iout_hbm, idx_v, rows_v, sem):
    wid = lax.axis_index("s") * NC + lax.axis_index("c")
    r0 = wid * N_CHUNK

    pltpu.sync_copy(uid_hbm.at[pl.ds(r0, N_CHUNK)], idx_v)
    cps = [pltpu.async_copy(utab_hbm.at[idx_v.at[j]], rows_v.at[j], sem)
           for j in range(N_CHUNK)]
    for cp in cps:
        cp.wait()
    pltpu.sync_copy(rows_v, uout_hbm.at[pl.ds(r0, N_CHUNK)])

    pltpu.sync_copy(iid_hbm.at[pl.ds(r0, N_CHUNK)], idx_v)
    cps = [pltpu.async_copy(itab_hbm.at[idx_v.at[j]], rows_v.at[j], sem)
           for j in range(N_CHUNK)]
    for cp in cps:
        cp.wait()
    pltpu.sync_copy(rows_v, iout_hbm.at[pl.ds(r0, N_CHUNK)])


def _sc_gather(user_ids2d, item_ids2d, user_table, item_table):
    mesh = plsc.VectorSubcoreMesh(core_axis_name="c", subcore_axis_name="s",
                                  num_cores=NC, num_subcores=NS)
    out_t = jax.ShapeDtypeStruct((N_IDX_ROWS, CHUNK, ED), jnp.float32)
    f = pl.kernel(
        _sc_gather_body,
        out_type=(out_t, out_t),
        mesh=mesh,
        scratch_types=[
            pltpu.VMEM((N_CHUNK, CHUNK), jnp.int32),
            pltpu.VMEM((N_CHUNK, CHUNK, ED), jnp.float32),
            pltpu.SemaphoreType.DMA,
        ],
    )
    return f(user_ids2d, item_ids2d, user_table, item_table)


def _mlp_body(x_ref, u_ref, i_ref, w1_ref, b1_ref, w2_ref, b2_ref,
              w3_ref, b3_ref, w4_ref, b4_ref, o_ref):
    bf = jnp.bfloat16
    f32 = jnp.float32
    x = x_ref[...].astype(bf)
    h = jnp.maximum(
        jnp.dot(x, w1_ref[...].astype(bf),
                preferred_element_type=f32) + b1_ref[...], 0.0)
    c = (jnp.dot(h.astype(bf), w2_ref[...].astype(bf),
                 preferred_element_type=f32) + b2_ref[...]).astype(bf)
    acc = (jnp.dot(u_ref[...].astype(bf), w3_ref[0:ED, :].astype(bf),
                   preferred_element_type=f32)
           + jnp.dot(i_ref[...].astype(bf), w3_ref[ED:2 * ED, :].astype(bf),
                     preferred_element_type=f32)
           + jnp.dot(c, w3_ref[2 * ED:3 * ED, :].astype(bf),
                     preferred_element_type=f32)
           + b3_ref[...])
    p = jnp.maximum(acc, 0.0)
    z = jnp.dot(p.astype(bf), w4_ref[...].astype(bf),
                preferred_element_type=f32) + b4_ref[...]
    o_ref[...] = jax.nn.sigmoid(z)


def _mlp(x, u, i, W1, b1, W2, b2, W3, b3, W4, b4, bs=2048):
    nblk = B // bs
    row_blk = lambda idx: (idx, 0)
    whole = lambda idx: (0, 0)
    return pl.pallas_call(
        _mlp_body,
        grid=(nblk,),
        in_specs=[
            pl.BlockSpec((bs, NF), row_blk),
            pl.BlockSpec((bs, ED), row_blk),
            pl.BlockSpec((bs, ED), row_blk),
            pl.BlockSpec((NF, ED), whole),
            pl.BlockSpec((1, ED), whole),
            pl.BlockSpec((ED, ED), whole),
            pl.BlockSpec((1, ED), whole),
            pl.BlockSpec((3 * ED, ED), whole),
            pl.BlockSpec((1, ED), whole),
            pl.BlockSpec((ED, 1), whole),
            pl.BlockSpec((1, 1), whole),
        ],
        out_specs=pl.BlockSpec((bs, 1), row_blk),
        out_shape=jax.ShapeDtypeStruct((B, 1), jnp.float32),
    )(x, u, i, W1, b1.reshape(1, ED), W2, b2.reshape(1, ED),
      W3, b3.reshape(1, ED), W4, b4.reshape(1, 1))


def kernel(user_ids, item_ids, item_features, user_table, item_table,
           W1, b1, W2, b2, W3, b3, W4, b4):
    uid2 = user_ids.astype(jnp.int32).reshape(N_IDX_ROWS, CHUNK)
    iid2 = item_ids.astype(jnp.int32).reshape(N_IDX_ROWS, CHUNK)
    u3, i3 = _sc_gather(uid2, iid2, user_table, item_table)
    u = u3.reshape(B, ED)
    i = i3.reshape(B, ED)
    return _mlp(item_features, u, i, W1, b1, W2, b2, W3, b3, W4, b4)
